# Initial kernel scaffold; baseline (speedup 1.0000x reference)
import jax
import jax.numpy as jnp
from jax.experimental import pallas as pl


def kernel(inputs, word_embedding, pos_embedding):
    raise NotImplementedError("write your pallas kernel here")


# speedup vs baseline: 1.7974x; 1.7974x over previous
import functools
import jax
import jax.numpy as jnp
from jax import lax
from jax.experimental import pallas as pl
from jax.experimental.pallas import tpu as pltpu
from jax.experimental.pallas import tpu_sc as plsc

NUM_POS = 50
EMBED_DIM = 64
CHUNK = 128
NUM_WORKERS = 32


def _build(total_rows):
    rows_per_worker = total_rows // NUM_WORKERS
    n_chunks = rows_per_worker // CHUNK
    mesh = plsc.VectorSubcoreMesh(
        core_axis_name="c", subcore_axis_name="s", num_cores=2, num_subcores=16
    )

    @functools.partial(
        pl.kernel,
        out_type=jax.ShapeDtypeStruct((total_rows, EMBED_DIM), jnp.float32),
        mesh=mesh,
        scratch_types=[
            pltpu.VMEM((n_chunks, CHUNK), jnp.int32),
            pltpu.VMEM((100, EMBED_DIM), jnp.float32),
            pltpu.VMEM((CHUNK, EMBED_DIM), jnp.float32),
            pltpu.SemaphoreType.DMA,
            pltpu.SemaphoreType.DMA,
        ],
        compiler_params=pltpu.CompilerParams(use_tc_tiling_on_sc=False),
    )
    def k(idx_hbm, table_hbm, pos_hbm, out_hbm, idx_v, pos_v, buf, gsem, osem):
        wid = lax.axis_index("s") * 2 + lax.axis_index("c")
        base = wid * rows_per_worker

        pltpu.sync_copy(pos_hbm, pos_v)
        pltpu.sync_copy(idx_hbm.at[wid], idx_v)

        def chunk_body(c, _):
            pltpu.async_copy(table_hbm.at[idx_v.at[c]], buf, gsem).wait()

            def row_body(r, _):
                l = lax.rem(c * CHUNK + r, NUM_POS)
                for j in range(EMBED_DIM // 16):
                    sl = pl.ds(j * 16, 16)
                    buf[r, sl] += pos_v[l, sl]
                return 0

            lax.fori_loop(0, CHUNK, row_body, 0)
            pltpu.async_copy(
                buf, out_hbm.at[pl.ds(base + c * CHUNK, CHUNK)], osem
            ).wait()
            return 0

        lax.fori_loop(0, n_chunks, chunk_body, 0)

    return k


def kernel(inputs, word_embedding, pos_embedding):
    B, L = inputs.shape
    total = B * L
    idx = inputs.astype(jnp.int32).reshape(NUM_WORKERS, total // NUM_WORKERS // CHUNK, CHUNK)
    out = _build(total)(idx, word_embedding, pos_embedding)
    return out.reshape(B, L, EMBED_DIM)


# speedup vs baseline: 2.1097x; 1.1738x over previous
import functools
import jax
import jax.numpy as jnp
from jax import lax
from jax.experimental import pallas as pl
from jax.experimental.pallas import tpu as pltpu
from jax.experimental.pallas import tpu_sc as plsc

SEQ = 50
EMBED_DIM = 64
CHUNK = 100
NBUF = 4
NUM_WORKERS = 32
LANES = 16


def _build(total_rows):
    rows_per_worker = total_rows // NUM_WORKERS
    n_chunks = rows_per_worker // CHUNK
    n_groups = n_chunks // NBUF
    mesh = plsc.VectorSubcoreMesh(
        core_axis_name="c", subcore_axis_name="s", num_cores=2, num_subcores=16
    )

    @functools.partial(
        pl.kernel,
        out_type=jax.ShapeDtypeStruct((total_rows, EMBED_DIM), jnp.float32),
        mesh=mesh,
        scratch_types=[
            pltpu.VMEM((n_chunks, CHUNK), jnp.int32),
            pltpu.VMEM((CHUNK, EMBED_DIM), jnp.float32),
            pltpu.VMEM((NBUF, CHUNK, EMBED_DIM), jnp.float32),
            pltpu.SemaphoreType.DMA((NBUF,)),
            pltpu.SemaphoreType.DMA((NBUF,)),
        ],
        compiler_params=pltpu.CompilerParams(use_tc_tiling_on_sc=False),
    )
    def k(idx_hbm, table_hbm, pos_hbm, out_hbm, idx_v, pos_v, bufs, gsems, osems):
        wid = lax.axis_index("s") * 2 + lax.axis_index("c")
        base = wid * rows_per_worker

        pltpu.sync_copy(pos_hbm, pos_v)
        pltpu.sync_copy(idx_hbm.at[wid], idx_v)

        for b in range(NBUF):
            pltpu.async_copy(table_hbm.at[idx_v.at[b]], bufs.at[b], gsems.at[b])

        def add_pos(b):
            def row_body(r, _):
                for j in range(EMBED_DIM // LANES):
                    sl = pl.ds(j * LANES, LANES)
                    plsc.addupdate(bufs.at[b, r, sl], pos_v[r, sl])
                return 0

            lax.fori_loop(0, CHUNK, row_body, 0)

        def group_body(g, _):
            for b in range(NBUF):
                c = g * NBUF + b
                pb = (b - 1) % NBUF
                pc = c - 1 + NBUF

                @pl.when(jnp.logical_and(pc >= NBUF, pc < n_chunks))
                def _():
                    pltpu.make_async_copy(
                        bufs.at[pb],
                        out_hbm.at[pl.ds(base + (pc - NBUF) * CHUNK, CHUNK)],
                        osems.at[pb],
                    ).wait()
                    pltpu.async_copy(
                        table_hbm.at[idx_v.at[pc]], bufs.at[pb], gsems.at[pb]
                    )

                pltpu.make_async_copy(
                    table_hbm.at[idx_v.at[b]], bufs.at[b], gsems.at[b]
                ).wait()
                add_pos(b)
                pltpu.async_copy(
                    bufs.at[b],
                    out_hbm.at[pl.ds(base + c * CHUNK, CHUNK)],
                    osems.at[b],
                )
            return 0

        lax.fori_loop(0, n_groups, group_body, 0)

        for b in range(NBUF):
            c = n_chunks - NBUF + b
            pltpu.make_async_copy(
                bufs.at[b],
                out_hbm.at[pl.ds(base + c * CHUNK, CHUNK)],
                osems.at[b],
            ).wait()

    return k


def kernel(inputs, word_embedding, pos_embedding):
    B, L = inputs.shape
    total = B * L
    idx = inputs.astype(jnp.int32).reshape(
        NUM_WORKERS, total // NUM_WORKERS // CHUNK, CHUNK
    )
    pos_tiled = jnp.tile(pos_embedding[:L], (CHUNK // L, 1))
    out = _build(total)(idx, word_embedding, pos_tiled)
    return out.reshape(B, L, EMBED_DIM)
